# baseline (speedup 1.0000x reference)
import functools

import jax
import jax.numpy as jnp
from jax import lax
from jax.experimental import pallas as pl
from jax.experimental.pallas import tpu as pltpu
from jax.experimental.pallas import tpu_sc as plsc

_NC = 2
_NS = 16
_L = 16
_B = 8
_C = 256
_CS = 128


def _layout(s, sp):
    wt = _NC * _NS
    assert s % 8 == 0
    q = (s // wt) // 8 * 8
    r = (s - q * wt) // 8
    swmax = q + (8 if r else 0)
    garb = swmax
    accr = ((swmax + 2 + _L - 1) // _L) * _L
    ow = ((swmax + 1 + 7) // 8) * 8
    assert ow <= sp
    steps = []
    st = 1
    while st < ow:
        st *= 2
    while st >= 1:
        steps.append(st)
        st //= 2
    return q, r, swmax, garb, accr, ow, steps


def _searcher(offs_l, ow, steps):
    def search(g):
        pos = jnp.zeros((_L,), jnp.int32)
        for stp in steps:
            cand = pos + stp
            cc = jnp.minimum(cand, ow - 1)
            v = plsc.load_gather(offs_l, [cc])
            pos = jnp.where((cand <= ow - 1) & (v <= g), cand, pos)
        return pos
    return search


@functools.lru_cache(maxsize=None)
def _build_bs(nb, d):
    tb = 512
    while nb % tb or tb % 8:
        tb -= 1

    def body(x_ref, o_ref):
        o_ref[...] = jnp.sum(x_ref[...], axis=1)

    return pl.pallas_call(
        body,
        grid=(nb // tb,),
        in_specs=[pl.BlockSpec((tb, _B, d), lambda i: (i, 0, 0))],
        out_specs=pl.BlockSpec((tb, d), lambda i: (i, 0)),
        out_shape=jax.ShapeDtypeStruct((nb, d), jnp.float32),
    )


@functools.lru_cache(maxsize=None)
def _build_edges(n, d, s, sp):
    q, r, swmax, garb, accr, ow, steps = _layout(s, sp)
    ng = (swmax + 1 + _L - 1) // _L
    mesh = plsc.VectorSubcoreMesh(core_axis_name="c", subcore_axis_name="s")

    @functools.partial(
        pl.kernel,
        out_type=jax.ShapeDtypeStruct((s, d), jnp.float32),
        mesh=mesh,
        scratch_types=[
            pltpu.VMEM((ow,), jnp.int32),
            pltpu.VMEM((_CS,), jnp.int32),
            pltpu.VMEM((_L,), jnp.int32),
            pltpu.VMEM((_CS, d), jnp.float32),
            pltpu.VMEM((_CS,), jnp.int32),
            pltpu.VMEM((_L, d), jnp.float32),
            pltpu.VMEM_SHARED((_NS * accr, d), jnp.float32),
        ],
        compiler_params=pltpu.CompilerParams(needs_layout_passes=False),
    )
    def edge_kernel(data_hbm, offs_hbm, oute_hbm, offs_l, ridx, kbuf, ebuf,
                    eidx, zbuf, acc):
        cid = lax.axis_index("c")
        sid = lax.axis_index("s")
        w = cid * _NS + sid
        s0 = w * q + jnp.minimum(w, r) * 8
        nseg = jnp.where(w < r, q + 8, q)
        abase = sid * accr

        for rr in range(_L):
            for cc in range(d // _L):
                zbuf[rr, pl.ds(cc * _L, _L)] = jnp.zeros((_L,), jnp.float32)
        for t in range(accr // _L):
            pltpu.sync_copy(zbuf, acc.at[pl.ds(abase + t * _L, _L)])

        base_a = jnp.minimum(s0, sp - ow)
        pltpu.sync_copy(offs_hbm.at[pl.ds(base_a, ow)], offs_l)
        search = _searcher(offs_l, ow, steps)
        lanes = lax.iota(jnp.int32, _L)
        nbm1 = n // _B - 1
        rs = plsc.load_gather(
            offs_l, [jnp.full((_L,), s0 - base_a, jnp.int32)])[0]

        def egroup(gi, carry):
            i = gi * _L + lanes
            inb = i <= nseg
            iw = jnp.minimum(s0 - base_a + i, ow - 1)
            bval = plsc.load_gather(offs_l, [iw])
            bvalp = plsc.load_gather(offs_l, [jnp.maximum(iw - 1, 0)])
            blk = bval // _B
            strad = (bval % _B) != 0
            dup = (blk == bvalp // _B) & ((bvalp % _B) != 0) & (i > 0)
            keep = inb & strad & jnp.logical_not(dup)
            kbuf[...] = keep.astype(jnp.int32)
            blkc = jnp.minimum(blk, nbm1)
            for j in range(_B):
                plsc.store_scatter(ridx, [lanes * _B + j], blkc * _B + j)
            pltpu.sync_copy(data_hbm.at[ridx], ebuf)
            for k in range(_CS // _L):
                g = ridx[pl.ds(k * _L, _L)]
                seg = base_a + search(g)
                kr = plsc.load_gather(kbuf, [(lanes // _B) + (_L // _B) * k])
                valid = (kr != 0) & (g >= rs) & (seg >= s0) & (seg < s0 + nseg)
                eidx[pl.ds(k * _L, _L)] = (
                    jnp.where(valid, seg - s0, garb) + abase)
            pltpu.sync_copy(ebuf, acc.at[eidx], add=True)
            return carry

        lax.fori_loop(0, ng, egroup, 0)

        if r:
            @pl.when(w < r)
            def _():
                pltpu.sync_copy(acc.at[pl.ds(abase, q + 8)],
                                oute_hbm.at[pl.ds(s0, q + 8)])

        if q:
            @pl.when(w >= r)
            def _():
                pltpu.sync_copy(acc.at[pl.ds(abase, q)],
                                oute_hbm.at[pl.ds(s0, q)])

    return edge_kernel


@functools.lru_cache(maxsize=None)
def _build_blocks(n, d, s, sp):
    q, r, swmax, garb, accr, ow, steps = _layout(s, sp)
    nb = n // _B
    mesh = plsc.VectorSubcoreMesh(core_axis_name="c", subcore_axis_name="s")

    @functools.partial(
        pl.kernel,
        out_type=jax.ShapeDtypeStruct((s, d), jnp.float32),
        mesh=mesh,
        scratch_types=[
            pltpu.VMEM((ow,), jnp.int32),
            [pltpu.VMEM((_C, d), jnp.float32) for _ in range(2)],
            [pltpu.VMEM((_CS,), jnp.int32) for _ in range(_C // _CS)],
            pltpu.VMEM((_L, d), jnp.float32),
            pltpu.VMEM_SHARED((_NS * accr, d), jnp.float32),
            [pltpu.SemaphoreType.DMA for _ in range(2)],
        ],
        compiler_params=pltpu.CompilerParams(needs_layout_passes=False),
    )
    def block_kernel(bs_hbm, offs_hbm, oute_hbm, out_hbm, offs_l, bufs,
                     idxbs, zbuf, acc, gsems):
        cid = lax.axis_index("c")
        sid = lax.axis_index("s")
        w = cid * _NS + sid
        s0 = w * q + jnp.minimum(w, r) * 8
        nseg = jnp.where(w < r, q + 8, q)
        abase = sid * accr

        for rr in range(_L):
            for cc in range(d // _L):
                zbuf[rr, pl.ds(cc * _L, _L)] = jnp.zeros((_L,), jnp.float32)
        for t in range(accr // _L):
            pltpu.sync_copy(zbuf, acc.at[pl.ds(abase + t * _L, _L)])
        if r:
            @pl.when(w < r)
            def _():
                pltpu.sync_copy(oute_hbm.at[pl.ds(s0, q + 8)],
                                acc.at[pl.ds(abase, q + 8)])
        if q:
            @pl.when(w >= r)
            def _():
                pltpu.sync_copy(oute_hbm.at[pl.ds(s0, q)],
                                acc.at[pl.ds(abase, q)])

        base_a = jnp.minimum(s0, sp - ow)
        pltpu.sync_copy(offs_hbm.at[pl.ds(base_a, ow)], offs_l)
        search = _searcher(offs_l, ow, steps)
        lanes = lax.iota(jnp.int32, _L)

        def _scalar_at(i):
            return plsc.load_gather(
                offs_l, [jnp.full((_L,), i, jnp.int32)])[0]

        rs = _scalar_at(s0 - base_a)
        re = _scalar_at(s0 + nseg - base_a)
        blo = (rs + _B - 1) // _B
        bhi = re // _B
        b8 = (blo // 8) * 8
        nch = jnp.maximum(bhi - b8 + (_C - 1), 0) // _C

        def _gather(c, buf, sem):
            base = b8 + c * _C
            cb = jnp.minimum(base, nb - _C)
            return pltpu.make_async_copy(bs_hbm.at[pl.ds(cb, _C)], buf, sem)

        def _process(c, buf):
            base = b8 + c * _C
            cb = jnp.minimum(base, nb - _C)
            for h in range(_C // _CS):
                idxb = idxbs[h]
                for gj in range(_CS // _L):
                    b = cb + h * _CS + gj * _L + lanes
                    valid = (b >= jnp.maximum(base, blo)) & (b < bhi)
                    pos1 = search(b * _B)
                    pos2 = search(b * _B + (_B - 1))
                    interior = pos1 == pos2
                    idxb[pl.ds(gj * _L, _L)] = (
                        jnp.where(valid & interior,
                                  base_a + pos1 - s0, garb) + abase)
                pltpu.sync_copy(buf.at[pl.ds(h * _CS, _CS)],
                                acc.at[idxb], add=True)

        @pl.when(nch > 0)
        def _():
            _gather(0, bufs[0], gsems[0]).start()

        def chunk_pair(c2, carry):
            for bnum in range(2):
                cc = 2 * c2 + bnum

                @pl.when(cc < nch)
                def _(cc=cc, bnum=bnum):
                    _gather(cc, bufs[bnum], gsems[bnum]).wait()
                    nxt = 1 - bnum

                    @pl.when(cc + 1 < nch)
                    def _():
                        _gather(cc + 1, bufs[nxt], gsems[nxt]).start()

                    _process(cc, bufs[bnum])
            return carry

        lax.fori_loop(0, (nch + 1) // 2, chunk_pair, 0)

        if r:
            @pl.when(w < r)
            def _():
                pltpu.sync_copy(acc.at[pl.ds(abase, q + 8)],
                                out_hbm.at[pl.ds(s0, q + 8)])

        if q:
            @pl.when(w >= r)
            def _():
                pltpu.sync_copy(acc.at[pl.ds(abase, q)],
                                out_hbm.at[pl.ds(s0, q)])

    return block_kernel


def kernel(data, offsets):
    n, d = data.shape
    s = offsets.shape[0] - 1
    offs = offsets.astype(jnp.int32)
    pad = (-offsets.shape[0]) % 8
    if pad:
        offs = jnp.concatenate([offs, jnp.full((pad,), n, jnp.int32)])
    sp = int(offs.shape[0])
    oute = _build_edges(n, d, s, sp)(data, offs)
    bs = _build_bs(n // _B, d)(data.reshape(n // _B, _B, d))
    return _build_blocks(n, d, s, sp)(bs, offs, oute)
